# Initial kernel scaffold; baseline (speedup 1.0000x reference)
import jax
import jax.numpy as jnp
from jax.experimental import pallas as pl


def kernel(inputs, anchors):
    raise NotImplementedError("write your pallas kernel here")


# speedup vs baseline: 1.9069x; 1.9069x over previous
import jax
import jax.numpy as jnp
from jax import lax
from jax.experimental import pallas as pl
from jax.experimental.pallas import tpu as pltpu

_NUM_CLASSES = 80


def _decode_body(anch_ref, in_ref, out_ref):
    t = in_ref[0, 0]
    n = t.shape[1]
    g = 52

    a = pl.program_id(1)
    aw = anch_ref[2 * a]
    ah = anch_ref[2 * a + 1]

    xy = t[0:2, :]
    wh = t[2:4, :]
    rest = t[4:, :]

    j = lax.broadcasted_iota(jnp.int32, (2, n), 1)
    row = lax.broadcasted_iota(jnp.int32, (2, n), 0)
    off = jnp.where(row == 0, j % g, j // g).astype(jnp.float32)
    abs_xy = (jax.nn.sigmoid(xy) + off) * (1.0 / g)

    arow = lax.broadcasted_iota(jnp.int32, (2, n), 0)
    anch = jnp.where(arow == 0, aw, ah)
    abs_wh = jnp.exp(wh) * anch

    res = jnp.concatenate([abs_xy, abs_wh, jax.nn.sigmoid(rest)], axis=0)
    out_ref[0, 0] = res.T


def kernel(inputs, anchors):
    B = inputs.shape[0]
    G = inputs.shape[2]
    A = anchors.shape[0]
    C5 = _NUM_CLASSES + 5
    N = G * G

    x = inputs.reshape(B, A, C5, N)
    anch_flat = anchors.reshape(-1)

    grid_spec = pltpu.PrefetchScalarGridSpec(
        num_scalar_prefetch=1,
        grid=(B, A),
        in_specs=[
            pl.BlockSpec((1, 1, C5, N), lambda b, a, s: (b, a, 0, 0)),
        ],
        out_specs=pl.BlockSpec((1, 1, N, C5), lambda b, a, s: (b, a, 0, 0)),
    )

    out = pl.pallas_call(
        _decode_body,
        grid_spec=grid_spec,
        out_shape=jax.ShapeDtypeStruct((B, A, N, C5), jnp.float32),
    )(anch_flat, x)

    return out.reshape(B, A * N, C5)


# speedup vs baseline: 1.9145x; 1.0039x over previous
import jax
import jax.numpy as jnp
from jax import lax
from jax.experimental import pallas as pl
from jax.experimental.pallas import tpu as pltpu

_NUM_CLASSES = 80


def _decode_body(anch_ref, in_ref, out_ref):
    t = in_ref[0, 0]
    n = t.shape[1]
    g = 52

    a = pl.program_id(1)
    aw = anch_ref[2 * a]
    ah = anch_ref[2 * a + 1]

    xy = t[0:2, :]
    wh = t[2:4, :]
    rest = t[4:, :]

    j = lax.broadcasted_iota(jnp.int32, (2, n), 1)
    row = lax.broadcasted_iota(jnp.int32, (2, n), 0)
    off = jnp.where(row == 0, j % g, j // g).astype(jnp.float32)
    abs_xy = (jax.nn.sigmoid(xy) + off) * (1.0 / g)

    arow = lax.broadcasted_iota(jnp.int32, (2, n), 0)
    anch = jnp.where(arow == 0, aw, ah)
    abs_wh = jnp.exp(wh) * anch

    res = jnp.concatenate([abs_xy, abs_wh, jax.nn.sigmoid(rest)], axis=0)
    out_ref[0] = res.T


def kernel(inputs, anchors):
    B = inputs.shape[0]
    G = inputs.shape[2]
    A = anchors.shape[0]
    C5 = _NUM_CLASSES + 5
    N = G * G

    x = inputs.reshape(B, A, C5, N)
    anch_flat = anchors.reshape(-1)

    grid_spec = pltpu.PrefetchScalarGridSpec(
        num_scalar_prefetch=1,
        grid=(B, A),
        in_specs=[
            pl.BlockSpec((1, 1, C5, N), lambda b, a, s: (b, a, 0, 0)),
        ],
        out_specs=pl.BlockSpec((1, N, C5), lambda b, a, s: (b, a, 0)),
    )

    out = pl.pallas_call(
        _decode_body,
        grid_spec=grid_spec,
        out_shape=jax.ShapeDtypeStruct((B, A * N, C5), jnp.float32),
    )(anch_flat, x)

    return out


# speedup vs baseline: 2.6505x; 1.3845x over previous
import jax
import jax.numpy as jnp
from jax import lax
from jax.experimental import pallas as pl
from jax.experimental.pallas import tpu as pltpu

_NUM_CLASSES = 80


def _decode_body(anch_ref, in_ref, out_ref):
    t3 = in_ref[0]
    g = t3.shape[1]
    n = g * g

    a = pl.program_id(1)
    aw = anch_ref[2 * a]
    ah = anch_ref[2 * a + 1]

    xy = t3[0:2]
    wh = t3[2:4]
    rest = t3[4:]

    chan = lax.broadcasted_iota(jnp.int32, (2, g, g), 0)
    gx = lax.broadcasted_iota(jnp.int32, (2, g, g), 2)
    gy = lax.broadcasted_iota(jnp.int32, (2, g, g), 1)
    off = jnp.where(chan == 0, gx, gy).astype(jnp.float32)
    abs_xy = (jax.nn.sigmoid(xy) + off) * (1.0 / g)

    anch = jnp.where(chan == 0, aw, ah)
    abs_wh = jnp.exp(wh) * anch

    res = jnp.concatenate([abs_xy, abs_wh, jax.nn.sigmoid(rest)], axis=0)
    out_ref[0] = jnp.transpose(res, (1, 2, 0)).reshape(n, res.shape[0])


def kernel(inputs, anchors):
    B = inputs.shape[0]
    G = inputs.shape[2]
    A = anchors.shape[0]
    C5 = _NUM_CLASSES + 5
    N = G * G

    anch_flat = anchors.reshape(-1)

    grid_spec = pltpu.PrefetchScalarGridSpec(
        num_scalar_prefetch=1,
        grid=(B, A),
        in_specs=[
            pl.BlockSpec((1, C5, G, G), lambda b, a, s: (b, a, 0, 0)),
        ],
        out_specs=pl.BlockSpec((1, N, C5), lambda b, a, s: (b, a, 0)),
    )

    return pl.pallas_call(
        _decode_body,
        grid_spec=grid_spec,
        out_shape=jax.ShapeDtypeStruct((B, A * N, C5), jnp.float32),
    )(anch_flat, inputs)


# speedup vs baseline: 2.8248x; 1.0658x over previous
import jax
import jax.numpy as jnp
from jax import lax
from jax.experimental import pallas as pl
from jax.experimental.pallas import tpu as pltpu

_NUM_CLASSES = 80


def _decode_body(anch_ref, in_ref, out_ref):
    t3 = in_ref[0]
    g = t3.shape[1]
    n = g * g

    a = pl.program_id(1)
    aw = anch_ref[2 * a]
    ah = anch_ref[2 * a + 1]

    xy = t3[0:2]
    wh = t3[2:4]
    rest = t3[4:]

    chan = lax.broadcasted_iota(jnp.int32, (2, g, g), 0)
    gx = lax.broadcasted_iota(jnp.int32, (2, g, g), 2)
    gy = lax.broadcasted_iota(jnp.int32, (2, g, g), 1)
    off = jnp.where(chan == 0, gx, gy).astype(jnp.float32)
    abs_xy = (jax.nn.sigmoid(xy) + off) * (1.0 / g)

    anch = jnp.where(chan == 0, aw, ah)
    abs_wh = jnp.exp(wh) * anch

    res = jnp.concatenate([abs_xy, abs_wh, jax.nn.sigmoid(rest)], axis=0)
    u = jnp.transpose(res, (1, 0, 2))
    v = jnp.transpose(u, (0, 2, 1))
    out_ref[0] = v.reshape(n, res.shape[0])


def kernel(inputs, anchors):
    B = inputs.shape[0]
    G = inputs.shape[2]
    A = anchors.shape[0]
    C5 = _NUM_CLASSES + 5
    N = G * G

    anch_flat = anchors.reshape(-1)

    grid_spec = pltpu.PrefetchScalarGridSpec(
        num_scalar_prefetch=1,
        grid=(B, A),
        in_specs=[
            pl.BlockSpec((1, C5, G, G), lambda b, a, s: (b, a, 0, 0)),
        ],
        out_specs=pl.BlockSpec((1, N, C5), lambda b, a, s: (b, a, 0)),
    )

    return pl.pallas_call(
        _decode_body,
        grid_spec=grid_spec,
        out_shape=jax.ShapeDtypeStruct((B, A * N, C5), jnp.float32),
    )(anch_flat, inputs)


# speedup vs baseline: 4.2600x; 1.5081x over previous
import jax
import jax.numpy as jnp
from jax import lax
from jax.experimental import pallas as pl
from jax.experimental.pallas import tpu as pltpu

_NUM_CLASSES = 80


def _decode_body(anch_ref, in_ref, out_ref):
    yp = pl.program_id(0)
    v = in_ref[...]
    g = v.shape[1]
    b = v.shape[2]
    c5 = _NUM_CLASSES + 5

    lane = lax.broadcasted_iota(jnp.int32, (2, g, b, c5), 3)
    xf = lax.broadcasted_iota(jnp.int32, (2, g, b, c5), 1).astype(jnp.float32)
    y2 = lax.broadcasted_iota(jnp.int32, (2, g, b, c5), 0)
    yf = (y2 + 2 * yp).astype(jnp.float32)
    off = jnp.where(lane == 0, xf, yf)
    inv_g = 1.0 / g

    for a in range(v.shape[3] // c5):
        va = v[:, :, :, a * c5:(a + 1) * c5]
        sig = jax.nn.sigmoid(va)
        ex = sig / (1.0 - sig)
        aw = anch_ref[2 * a]
        ah = anch_ref[2 * a + 1]
        scale = jnp.where(lane == 2, aw, ah)
        dec = jnp.where(
            lane < 2, (sig + off) * inv_g,
            jnp.where(lane < 4, ex * scale, sig))
        out_ref[:, a] = jnp.transpose(dec, (2, 0, 1, 3)).reshape(b, 2 * g, c5)


def kernel(inputs, anchors):
    B = inputs.shape[0]
    G = inputs.shape[2]
    A = anchors.shape[0]
    C5 = _NUM_CLASSES + 5
    N = G * G

    xt = jnp.transpose(inputs, (2, 3, 0, 1))
    anch_flat = anchors.reshape(-1)

    grid_spec = pltpu.PrefetchScalarGridSpec(
        num_scalar_prefetch=1,
        grid=(G // 2,),
        in_specs=[
            pl.BlockSpec((2, G, B, A * C5), lambda yp, s: (yp, 0, 0, 0)),
        ],
        out_specs=pl.BlockSpec((B, A, 2 * G, C5), lambda yp, s: (0, 0, yp, 0)),
    )

    out = pl.pallas_call(
        _decode_body,
        grid_spec=grid_spec,
        out_shape=jax.ShapeDtypeStruct((B, A, N, C5), jnp.float32),
    )(anch_flat, xt)

    return out.reshape(B, A * N, C5)
